# baseline (speedup 1.0000x reference)
import functools

import jax
import jax.numpy as jnp
from jax import lax
from jax.experimental import pallas as pl
from jax.experimental.pallas import tpu as pltpu
from jax.experimental.pallas import tpu_sc as plsc

N = 10000
NP = 10240
E = 320000
D_IN = 128
D_HID = 128
D_OUT = 64
DH = 64

NC = 2
NS = 16
NW = NC * NS
CHUNK = 128
NCHUNK = E // CHUNK
CPT = 80
CPT_LAST = NCHUNK - (NW - 1) * CPT
RPS = NP // NS

_mesh = plsc.VectorSubcoreMesh(core_axis_name="c", subcore_axis_name="s")
_sc_params = pltpu.CompilerParams(use_tc_tiling_on_sc=False,
                                  needs_layout_passes=False)


@functools.partial(
    pl.kernel,
    out_type=(jax.ShapeDtypeStruct((NC, NP, 16), jnp.float32),
              jax.ShapeDtypeStruct((NC, NP, 16), jnp.float32)),
    mesh=_mesh,
    compiler_params=_sc_params,
    scratch_types=[
        pltpu.VMEM((CPT, CHUNK), jnp.int32),
        pltpu.VMEM((CPT, CHUNK), jnp.int32),
        pltpu.VMEM((CHUNK, 16), jnp.float32),
        pltpu.VMEM_SHARED((NP, 16), jnp.float32),
        pltpu.VMEM_SHARED((NP, 16), jnp.float32),
    ],
)
def _deg_kernel(ei_hbm, ones_hbm, zeros_hbm,
                hsrc_out, hdst_out, si, di, ones_v, hs_sh, hd_sh):
    src_hbm = ei_hbm.at[0]
    dst_hbm = ei_hbm.at[1]
    cid = lax.axis_index("c")
    sid = lax.axis_index("s")
    r0 = sid * RPS
    pltpu.sync_copy(zeros_hbm.at[pl.ds(r0, RPS)], hs_sh.at[pl.ds(r0, RPS)])
    pltpu.sync_copy(zeros_hbm.at[pl.ds(r0, RPS)], hd_sh.at[pl.ds(r0, RPS)])
    pltpu.sync_copy(ones_hbm, ones_v)
    wid = cid * NS + sid
    c0 = wid * CPT
    cpt = _load_idx(src_hbm, dst_hbm, si, di, wid, c0)
    plsc.subcore_barrier()

    @pl.loop(0, cpt)
    def _(i):
        pltpu.sync_copy(ones_v, hs_sh.at[si.at[i]], add=True)
        pltpu.sync_copy(ones_v, hd_sh.at[di.at[i]], add=True)

    plsc.subcore_barrier()
    pltpu.sync_copy(hs_sh.at[pl.ds(r0, RPS)], hsrc_out.at[cid, pl.ds(r0, RPS)])
    pltpu.sync_copy(hd_sh.at[pl.ds(r0, RPS)], hdst_out.at[cid, pl.ds(r0, RPS)])


NSLOT = 4


def _load_idx(src_hbm, dst_hbm, si, di, wid, c0):
    @pl.when(wid == NW - 1)
    def _():
        pltpu.sync_copy(src_hbm.at[pl.ds(c0, CPT_LAST)],
                        si.at[pl.ds(0, CPT_LAST)])
        pltpu.sync_copy(dst_hbm.at[pl.ds(c0, CPT_LAST)],
                        di.at[pl.ds(0, CPT_LAST)])

    @pl.when(wid != NW - 1)
    def _():
        pltpu.sync_copy(src_hbm.at[pl.ds(c0, CPT)], si)
        pltpu.sync_copy(dst_hbm.at[pl.ds(c0, CPT)], di)

    return lax.select(wid == NW - 1, CPT_LAST, CPT)


def _agg_phase(h_hbm, out_hbm, zeros_hbm, si, di, msgs, acc_sh,
               gsems, ssems, cpt, r0, cid):
    def gather(i, k):
        pltpu.make_async_copy(h_hbm.at[si.at[i]], msgs[k], gsems[k]).start()

    def wait_g(i, k):
        pltpu.make_async_copy(h_hbm.at[si.at[i]], msgs[k], gsems[k]).wait()

    def scat(i, k):
        pltpu.make_async_copy(msgs[k], acc_sh.at[di.at[i]],
                              ssems[k]).start(add=True)

    def wait_s(i, k):
        pltpu.make_async_copy(msgs[k], acc_sh.at[di.at[i]], ssems[k]).wait()

    for k in range(NSLOT):
        gather(k, k)
    pltpu.sync_copy(zeros_hbm.at[pl.ds(r0, RPS)], acc_sh.at[pl.ds(r0, RPS)])
    plsc.subcore_barrier()

    @pl.loop(0, cpt // NSLOT - 1)
    def _(j):
        i = NSLOT * j
        for k in range(NSLOT):
            wait_g(i + k, k)
            scat(i + k, k)
        for k in range(NSLOT):
            wait_s(i + k, k)
            gather(i + NSLOT + k, k)

    last = cpt - NSLOT
    for k in range(NSLOT):
        wait_g(last + k, k)
        scat(last + k, k)
    for k in range(NSLOT):
        wait_s(last + k, k)

    plsc.subcore_barrier()
    pltpu.sync_copy(acc_sh.at[pl.ds(r0, RPS)], out_hbm.at[cid, pl.ds(r0, RPS)])


_AGG_SCRATCH = (
    [pltpu.VMEM((CPT, CHUNK), jnp.int32),
     pltpu.VMEM((CPT, CHUNK), jnp.int32)]
    + [pltpu.VMEM((CHUNK, DH), jnp.float32)] * NSLOT
    + [pltpu.VMEM_SHARED((NP, DH), jnp.float32)]
    + [pltpu.SemaphoreType.DMA] * (2 * NSLOT)
)


SB = 64


@functools.partial(
    pl.kernel,
    out_type=(jax.ShapeDtypeStruct((NC, NP, DH), jnp.float32),
              jax.ShapeDtypeStruct((NC, NP, DH), jnp.float32),
              jax.ShapeDtypeStruct((NC, NP, DH), jnp.float32),
              jax.ShapeDtypeStruct((NC, NP, DH), jnp.float32)),
    mesh=_mesh,
    compiler_params=_sc_params,
    scratch_types=_AGG_SCRATCH + [
        pltpu.VMEM((SB, D_IN), jnp.float32),
        pltpu.VMEM((SB, 16), jnp.float32),
        pltpu.VMEM((SB, 16), jnp.float32),
        pltpu.VMEM((SB, DH), jnp.float32),
        pltpu.VMEM((SB, DH), jnp.float32),
    ],
)
def _scale_agg(x_hbm, hs_hbm, ei_hbm, zeros_hbm,
               h_lo, h_hi, out_lo, out_hi,
               si, di, m0, m1, m2, m3, acc_sh, g0, g1, g2, g3, s0, s1, s2, s3,
               x_blk, hs0_blk, hs1_blk, hlo_blk, hhi_blk):
    src_hbm = ei_hbm.at[0]
    dst_hbm = ei_hbm.at[1]
    msgs = [m0, m1, m2, m3]
    gsems = [g0, g1, g2, g3]
    ssems = [s0, s1, s2, s3]
    cid = lax.axis_index("c")
    sid = lax.axis_index("s")
    r0 = sid * RPS
    wid = cid * NS + sid
    cpt = _load_idx(src_hbm, dst_hbm, si, di, wid, wid * CPT)

    @pl.loop(0, RPS // SB)
    def _(b):
        rb = r0 + b * SB
        pltpu.sync_copy(x_hbm.at[pl.ds(rb, SB)], x_blk)
        pltpu.sync_copy(hs_hbm.at[0, pl.ds(rb, SB)], hs0_blk)
        pltpu.sync_copy(hs_hbm.at[1, pl.ds(rb, SB)], hs1_blk)

        @pl.loop(0, SB)
        def _(r):
            d = jnp.maximum(hs0_blk[r] + hs1_blk[r], 1.0)
            i = 0x5F3759DF - lax.shift_right_logical(plsc.bitcast(d, jnp.int32), 1)
            y = plsc.bitcast(i, jnp.float32)
            y = y * (1.5 - 0.5 * d * y * y)
            y = y * (1.5 - 0.5 * d * y * y)
            y = y * (1.5 - 0.5 * d * y * y)
            for c in range(4):
                hlo_blk[r, pl.ds(16 * c, 16)] = x_blk[r, pl.ds(16 * c, 16)] * y
            for c in range(4):
                hhi_blk[r, pl.ds(16 * c, 16)] = (
                    x_blk[r, pl.ds(DH + 16 * c, 16)] * y)

        pltpu.sync_copy(hlo_blk, h_lo.at[cid, pl.ds(rb, SB)])
        pltpu.sync_copy(hhi_blk, h_hi.at[cid, pl.ds(rb, SB)])

    plsc.subcore_barrier()
    _agg_phase(h_lo.at[cid], out_lo, zeros_hbm, si, di, msgs, acc_sh,
               gsems, ssems, cpt, r0, cid)
    plsc.subcore_barrier()
    _agg_phase(h_hi.at[cid], out_hi, zeros_hbm, si, di, msgs, acc_sh,
               gsems, ssems, cpt, r0, cid)


@functools.partial(
    pl.kernel,
    out_type=jax.ShapeDtypeStruct((NC, NP, DH), jnp.float32),
    mesh=_mesh,
    compiler_params=_sc_params,
    scratch_types=_AGG_SCRATCH,
)
def _agg64(h_hbm, ei_hbm, zeros_hbm, out_hbm,
           si, di, m0, m1, m2, m3, acc_sh, g0, g1, g2, g3, s0, s1, s2, s3):
    src_hbm = ei_hbm.at[0]
    dst_hbm = ei_hbm.at[1]
    msgs = [m0, m1, m2, m3]
    gsems = [g0, g1, g2, g3]
    ssems = [s0, s1, s2, s3]
    cid = lax.axis_index("c")
    sid = lax.axis_index("s")
    r0 = sid * RPS
    wid = cid * NS + sid
    cpt = _load_idx(src_hbm, dst_hbm, si, di, wid, wid * CPT)
    _agg_phase(h_hbm, out_hbm, zeros_hbm, si, di, msgs, acc_sh,
               gsems, ssems, cpt, r0, cid)


MMB = 1024


def _mm_body(a_lo, a_hi, hd, hs, w1, b1, w2, out):
    lo = a_lo[0] + a_lo[1]
    hi = a_hi[0] + a_hi[1]
    nd = lax.rsqrt(jnp.maximum(hd[0, :, 0:1] + hd[1, :, 0:1], 1.0))
    t = (jnp.dot(lo * nd, w1[0:DH, :], preferred_element_type=jnp.float32)
         + jnp.dot(hi * nd, w1[DH:D_IN, :], preferred_element_type=jnp.float32)
         + b1[...])
    t = jnp.maximum(t, 0.0)
    ns = lax.rsqrt(jnp.maximum(hs[0, :, 0:1] + hs[1, :, 0:1], 1.0))
    out[...] = jnp.dot(t * ns, w2[...], preferred_element_type=jnp.float32)


_mm = pl.pallas_call(
    _mm_body,
    grid=(NP // MMB,),
    in_specs=[
        pl.BlockSpec((2, MMB, DH), lambda r: (0, r, 0)),
        pl.BlockSpec((2, MMB, DH), lambda r: (0, r, 0)),
        pl.BlockSpec((2, MMB, 16), lambda r: (0, r, 0)),
        pl.BlockSpec((2, MMB, 16), lambda r: (0, r, 0)),
        pl.BlockSpec((D_IN, D_HID), lambda r: (0, 0)),
        pl.BlockSpec((1, D_HID), lambda r: (0, 0)),
        pl.BlockSpec((D_HID, D_OUT), lambda r: (0, 0)),
    ],
    out_specs=pl.BlockSpec((MMB, D_OUT), lambda r: (r, 0)),
    out_shape=jax.ShapeDtypeStruct((NP, D_OUT), jnp.float32),
)


FB = 1000


def _final_body(agg, hd, b2, out):
    a = agg[0] + agg[1]
    nd = lax.rsqrt(jnp.maximum(hd[0, :, 0:1] + hd[1, :, 0:1], 1.0))
    out[...] = a * nd + b2[...]


_final = pl.pallas_call(
    _final_body,
    grid=(N // FB,),
    in_specs=[
        pl.BlockSpec((2, FB, DH), lambda r: (0, r, 0)),
        pl.BlockSpec((2, FB, 16), lambda r: (0, r, 0)),
        pl.BlockSpec((1, D_OUT), lambda r: (0, 0)),
    ],
    out_specs=pl.BlockSpec((FB, D_OUT), lambda r: (r, 0)),
    out_shape=jax.ShapeDtypeStruct((N, D_OUT), jnp.float32),
)


def kernel(features, edge_index, W1, b1, W2, b2):
    ei3 = edge_index.astype(jnp.int32).reshape(2, NCHUNK, CHUNK)
    ones16 = jnp.ones((CHUNK, 16), jnp.float32)
    z16 = jnp.zeros((NP, 16), jnp.float32)
    z64 = jnp.zeros((NP, DH), jnp.float32)

    x_pad = jnp.pad(features, ((0, NP - N), (0, 0)))

    hsrc, hdst = _deg_kernel(ei3, ones16, z16)
    _hl, _hh, a_lo, a_hi = _scale_agg(x_pad, hsrc, ei3, z64)
    z = _mm(a_lo, a_hi, hdst, hsrc, W1, b1.reshape(1, D_HID), W2)
    agg2 = _agg64(z, ei3, z64)
    return _final(agg2, hdst, b2.reshape(1, D_OUT))


# speedup vs baseline: 1.1132x; 1.1132x over previous
import functools

import jax
import jax.numpy as jnp
from jax import lax
from jax.experimental import pallas as pl
from jax.experimental.pallas import tpu as pltpu
from jax.experimental.pallas import tpu_sc as plsc

N = 10000
NP = 10240
E = 320000
D_IN = 128
D_HID = 128
D_OUT = 64
DH = 64

NC = 2
NS = 16
NW = NC * NS
CHUNK = 128
NCHUNK = E // CHUNK
CPT = 80
CPT_LAST = NCHUNK - (NW - 1) * CPT
RPS = NP // NS

_mesh = plsc.VectorSubcoreMesh(core_axis_name="c", subcore_axis_name="s")
_sc_params = pltpu.CompilerParams(use_tc_tiling_on_sc=False)


@functools.partial(
    pl.kernel,
    out_type=(jax.ShapeDtypeStruct((NC, NP, 16), jnp.float32),
              jax.ShapeDtypeStruct((NC, NP, 16), jnp.float32)),
    mesh=_mesh,
    compiler_params=_sc_params,
    scratch_types=[
        pltpu.VMEM((CPT, CHUNK), jnp.int32),
        pltpu.VMEM((CPT, CHUNK), jnp.int32),
        pltpu.VMEM((CHUNK, 16), jnp.float32),
        pltpu.VMEM_SHARED((NP, 16), jnp.float32),
        pltpu.VMEM_SHARED((NP, 16), jnp.float32),
    ],
)
def _deg_kernel(ei_hbm, ones_hbm, zeros_hbm,
                hsrc_out, hdst_out, si, di, ones_v, hs_sh, hd_sh):
    src_hbm = ei_hbm.at[0]
    dst_hbm = ei_hbm.at[1]
    cid = lax.axis_index("c")
    sid = lax.axis_index("s")
    r0 = sid * RPS
    pltpu.sync_copy(zeros_hbm.at[pl.ds(r0, RPS)], hs_sh.at[pl.ds(r0, RPS)])
    pltpu.sync_copy(zeros_hbm.at[pl.ds(r0, RPS)], hd_sh.at[pl.ds(r0, RPS)])
    pltpu.sync_copy(ones_hbm, ones_v)
    wid = cid * NS + sid
    c0 = wid * CPT
    cpt = _load_idx(src_hbm, dst_hbm, si, di, wid, c0)
    plsc.subcore_barrier()

    @pl.loop(0, cpt)
    def _(i):
        pltpu.sync_copy(ones_v, hs_sh.at[si.at[i]], add=True)
        pltpu.sync_copy(ones_v, hd_sh.at[di.at[i]], add=True)

    plsc.subcore_barrier()
    pltpu.sync_copy(hs_sh.at[pl.ds(r0, RPS)], hsrc_out.at[cid, pl.ds(r0, RPS)])
    pltpu.sync_copy(hd_sh.at[pl.ds(r0, RPS)], hdst_out.at[cid, pl.ds(r0, RPS)])


NSLOT = 4


def _load_idx(src_hbm, dst_hbm, si, di, wid, c0):
    @pl.when(wid == NW - 1)
    def _():
        pltpu.sync_copy(src_hbm.at[pl.ds(c0, CPT_LAST)],
                        si.at[pl.ds(0, CPT_LAST)])
        pltpu.sync_copy(dst_hbm.at[pl.ds(c0, CPT_LAST)],
                        di.at[pl.ds(0, CPT_LAST)])

    @pl.when(wid != NW - 1)
    def _():
        pltpu.sync_copy(src_hbm.at[pl.ds(c0, CPT)], si)
        pltpu.sync_copy(dst_hbm.at[pl.ds(c0, CPT)], di)

    return lax.select(wid == NW - 1, CPT_LAST, CPT)


def _agg_phase(h_hbm, out_hbm, zeros_hbm, si, di, msgs, acc_sh,
               gsems, ssems, cpt, r0, cid):
    def gather(i, k):
        pltpu.make_async_copy(h_hbm.at[si.at[i]], msgs[k], gsems[k]).start()

    def wait_g(i, k):
        pltpu.make_async_copy(h_hbm.at[si.at[i]], msgs[k], gsems[k]).wait()

    def scat(i, k):
        pltpu.make_async_copy(msgs[k], acc_sh.at[di.at[i]],
                              ssems[k]).start(add=True)

    def wait_s(i, k):
        pltpu.make_async_copy(msgs[k], acc_sh.at[di.at[i]], ssems[k]).wait()

    for k in range(NSLOT):
        gather(k, k)
    pltpu.sync_copy(zeros_hbm.at[pl.ds(r0, RPS)], acc_sh.at[pl.ds(r0, RPS)])
    plsc.subcore_barrier()

    @pl.loop(0, cpt // NSLOT - 1)
    def _(j):
        i = NSLOT * j
        for k in range(NSLOT):
            wait_g(i + k, k)
            scat(i + k, k)
        for k in range(NSLOT):
            wait_s(i + k, k)
            gather(i + NSLOT + k, k)

    last = cpt - NSLOT
    for k in range(NSLOT):
        wait_g(last + k, k)
        scat(last + k, k)
    for k in range(NSLOT):
        wait_s(last + k, k)

    plsc.subcore_barrier()
    pltpu.sync_copy(acc_sh.at[pl.ds(r0, RPS)], out_hbm.at[cid, pl.ds(r0, RPS)])


_AGG_SCRATCH = (
    [pltpu.VMEM((CPT, CHUNK), jnp.int32),
     pltpu.VMEM((CPT, CHUNK), jnp.int32)]
    + [pltpu.VMEM((CHUNK, DH), jnp.float32)] * NSLOT
    + [pltpu.VMEM_SHARED((NP, DH), jnp.float32)]
    + [pltpu.SemaphoreType.DMA] * (2 * NSLOT)
)


@functools.partial(
    pl.kernel,
    out_type=(jax.ShapeDtypeStruct((NC, NP, DH), jnp.float32),
              jax.ShapeDtypeStruct((NC, NP, DH), jnp.float32)),
    mesh=_mesh,
    compiler_params=_sc_params,
    scratch_types=_AGG_SCRATCH,
)
def _agg2x64(h_lo, h_hi, ei_hbm, zeros_hbm, out_lo, out_hi,
             si, di, m0, m1, m2, m3, acc_sh, g0, g1, g2, g3, s0, s1, s2, s3):
    src_hbm = ei_hbm.at[0]
    dst_hbm = ei_hbm.at[1]
    msgs = [m0, m1, m2, m3]
    gsems = [g0, g1, g2, g3]
    ssems = [s0, s1, s2, s3]
    cid = lax.axis_index("c")
    sid = lax.axis_index("s")
    r0 = sid * RPS
    wid = cid * NS + sid
    cpt = _load_idx(src_hbm, dst_hbm, si, di, wid, wid * CPT)
    _agg_phase(h_lo, out_lo, zeros_hbm, si, di, msgs, acc_sh,
               gsems, ssems, cpt, r0, cid)
    plsc.subcore_barrier()
    _agg_phase(h_hi, out_hi, zeros_hbm, si, di, msgs, acc_sh,
               gsems, ssems, cpt, r0, cid)


@functools.partial(
    pl.kernel,
    out_type=jax.ShapeDtypeStruct((NC, NP, DH), jnp.float32),
    mesh=_mesh,
    compiler_params=_sc_params,
    scratch_types=_AGG_SCRATCH,
)
def _agg64(h_hbm, ei_hbm, zeros_hbm, out_hbm,
           si, di, m0, m1, m2, m3, acc_sh, g0, g1, g2, g3, s0, s1, s2, s3):
    src_hbm = ei_hbm.at[0]
    dst_hbm = ei_hbm.at[1]
    msgs = [m0, m1, m2, m3]
    gsems = [g0, g1, g2, g3]
    ssems = [s0, s1, s2, s3]
    cid = lax.axis_index("c")
    sid = lax.axis_index("s")
    r0 = sid * RPS
    wid = cid * NS + sid
    cpt = _load_idx(src_hbm, dst_hbm, si, di, wid, wid * CPT)
    _agg_phase(h_hbm, out_hbm, zeros_hbm, si, di, msgs, acc_sh,
               gsems, ssems, cpt, r0, cid)


def _scale_body(feat, hs, out_lo, out_hi):
    deg = hs[0, 0:N, 0:1] + hs[1, 0:N, 0:1]
    ns = lax.rsqrt(jnp.maximum(deg, 1.0))
    h = feat[...] * ns
    out_lo[0:N, :] = h[:, 0:DH]
    out_hi[0:N, :] = h[:, DH:D_IN]
    zero_tail = jnp.zeros((NP - N, DH), jnp.float32)
    out_lo[N:NP, :] = zero_tail
    out_hi[N:NP, :] = zero_tail


_scale = pl.pallas_call(
    _scale_body,
    out_shape=(jax.ShapeDtypeStruct((NP, DH), jnp.float32),
               jax.ShapeDtypeStruct((NP, DH), jnp.float32)),
)


MMB = 1024


def _mm_body(a_lo, a_hi, hd, hs, w1, b1, w2, out):
    lo = a_lo[0] + a_lo[1]
    hi = a_hi[0] + a_hi[1]
    nd = lax.rsqrt(jnp.maximum(hd[0, :, 0:1] + hd[1, :, 0:1], 1.0))
    t = (jnp.dot(lo * nd, w1[0:DH, :], preferred_element_type=jnp.float32)
         + jnp.dot(hi * nd, w1[DH:D_IN, :], preferred_element_type=jnp.float32)
         + b1[...])
    t = jnp.maximum(t, 0.0)
    ns = lax.rsqrt(jnp.maximum(hs[0, :, 0:1] + hs[1, :, 0:1], 1.0))
    out[...] = jnp.dot(t * ns, w2[...], preferred_element_type=jnp.float32)


_mm = pl.pallas_call(
    _mm_body,
    grid=(NP // MMB,),
    in_specs=[
        pl.BlockSpec((2, MMB, DH), lambda r: (0, r, 0)),
        pl.BlockSpec((2, MMB, DH), lambda r: (0, r, 0)),
        pl.BlockSpec((2, MMB, 16), lambda r: (0, r, 0)),
        pl.BlockSpec((2, MMB, 16), lambda r: (0, r, 0)),
        pl.BlockSpec((D_IN, D_HID), lambda r: (0, 0)),
        pl.BlockSpec((1, D_HID), lambda r: (0, 0)),
        pl.BlockSpec((D_HID, D_OUT), lambda r: (0, 0)),
    ],
    out_specs=pl.BlockSpec((MMB, D_OUT), lambda r: (r, 0)),
    out_shape=jax.ShapeDtypeStruct((NP, D_OUT), jnp.float32),
)


FB = 1000


def _final_body(agg, hd, b2, out):
    a = agg[0] + agg[1]
    nd = lax.rsqrt(jnp.maximum(hd[0, :, 0:1] + hd[1, :, 0:1], 1.0))
    out[...] = a * nd + b2[...]


_final = pl.pallas_call(
    _final_body,
    grid=(N // FB,),
    in_specs=[
        pl.BlockSpec((2, FB, DH), lambda r: (0, r, 0)),
        pl.BlockSpec((2, FB, 16), lambda r: (0, r, 0)),
        pl.BlockSpec((1, D_OUT), lambda r: (0, 0)),
    ],
    out_specs=pl.BlockSpec((FB, D_OUT), lambda r: (r, 0)),
    out_shape=jax.ShapeDtypeStruct((N, D_OUT), jnp.float32),
)


def kernel(features, edge_index, W1, b1, W2, b2):
    ei3 = edge_index.astype(jnp.int32).reshape(2, NCHUNK, CHUNK)
    ones16 = jnp.ones((CHUNK, 16), jnp.float32)
    z16 = jnp.zeros((NP, 16), jnp.float32)
    z64 = jnp.zeros((NP, DH), jnp.float32)

    hsrc, hdst = _deg_kernel(ei3, ones16, z16)
    h_lo, h_hi = _scale(features, hsrc)
    a_lo, a_hi = _agg2x64(h_lo, h_hi, ei3, z64)
    z = _mm(a_lo, a_hi, hdst, hsrc, W1, b1.reshape(1, D_HID), W2)
    agg2 = _agg64(z, ei3, z64)
    return _final(agg2, hdst, b2.reshape(1, D_OUT))


# speedup vs baseline: 1.1492x; 1.0324x over previous
import functools

import jax
import jax.numpy as jnp
from jax import lax
from jax.experimental import pallas as pl
from jax.experimental.pallas import tpu as pltpu
from jax.experimental.pallas import tpu_sc as plsc

N = 10000
NP = 10240
E = 320000
D_IN = 128
D_HID = 128
D_OUT = 64
DH = 64

NC = 2
NS = 16
NW = NC * NS
CHUNK = 128
NCHUNK = E // CHUNK
CPT = 80
CPT_LAST = NCHUNK - (NW - 1) * CPT
RPS = NP // NS

_mesh = plsc.VectorSubcoreMesh(core_axis_name="c", subcore_axis_name="s")
_sc_params = pltpu.CompilerParams(use_tc_tiling_on_sc=False)


@functools.partial(
    pl.kernel,
    out_type=(jax.ShapeDtypeStruct((NC, NP, 16), jnp.float32),
              jax.ShapeDtypeStruct((NC, NP, 16), jnp.float32)),
    mesh=_mesh,
    compiler_params=_sc_params,
    scratch_types=[
        pltpu.VMEM((CPT, CHUNK), jnp.int32),
        pltpu.VMEM((CPT, CHUNK), jnp.int32),
        pltpu.VMEM((CHUNK, 16), jnp.float32),
        pltpu.VMEM_SHARED((NP, 16), jnp.float32),
        pltpu.VMEM_SHARED((NP, 16), jnp.float32),
        pltpu.SemaphoreType.DMA,
        pltpu.SemaphoreType.DMA,
        pltpu.SemaphoreType.DMA,
        pltpu.SemaphoreType.DMA,
        pltpu.SemaphoreType.DMA,
        pltpu.SemaphoreType.DMA,
        pltpu.SemaphoreType.DMA,
        pltpu.SemaphoreType.DMA,
    ],
)
def _deg_kernel(ei_hbm, ones_hbm, zeros_hbm,
                hsrc_out, hdst_out, si, di, ones_v, hs_sh, hd_sh,
                a0, a1, a2, a3, b0, b1, b2, b3):
    src_hbm = ei_hbm.at[0]
    dst_hbm = ei_hbm.at[1]
    asems = [a0, a1, a2, a3]
    bsems = [b0, b1, b2, b3]
    cid = lax.axis_index("c")
    sid = lax.axis_index("s")
    r0 = sid * RPS
    pltpu.sync_copy(zeros_hbm.at[pl.ds(r0, RPS)], hs_sh.at[pl.ds(r0, RPS)])
    pltpu.sync_copy(zeros_hbm.at[pl.ds(r0, RPS)], hd_sh.at[pl.ds(r0, RPS)])
    pltpu.sync_copy(ones_hbm, ones_v)
    wid = cid * NS + sid
    c0 = wid * CPT
    cpt = _load_idx(src_hbm, dst_hbm, si, di, wid, c0)
    plsc.subcore_barrier()

    def s_hs(i, k):
        pltpu.make_async_copy(ones_v, hs_sh.at[si.at[i]],
                              asems[k]).start(add=True)

    def w_hs(i, k):
        pltpu.make_async_copy(ones_v, hs_sh.at[si.at[i]], asems[k]).wait()

    def s_hd(i, k):
        pltpu.make_async_copy(ones_v, hd_sh.at[di.at[i]],
                              bsems[k]).start(add=True)

    def w_hd(i, k):
        pltpu.make_async_copy(ones_v, hd_sh.at[di.at[i]], bsems[k]).wait()

    for k in range(NSLOT):
        s_hs(k, k)
        s_hd(k, k)

    @pl.loop(0, cpt // NSLOT - 1)
    def _(j):
        i = NSLOT * j
        for k in range(NSLOT):
            w_hs(i + k, k)
            s_hs(i + NSLOT + k, k)
            w_hd(i + k, k)
            s_hd(i + NSLOT + k, k)

    last = cpt - NSLOT
    for k in range(NSLOT):
        w_hs(last + k, k)
        w_hd(last + k, k)

    plsc.subcore_barrier()
    pltpu.sync_copy(hs_sh.at[pl.ds(r0, RPS)], hsrc_out.at[cid, pl.ds(r0, RPS)])
    pltpu.sync_copy(hd_sh.at[pl.ds(r0, RPS)], hdst_out.at[cid, pl.ds(r0, RPS)])


NSLOT = 4


def _load_idx(src_hbm, dst_hbm, si, di, wid, c0):
    @pl.when(wid == NW - 1)
    def _():
        pltpu.sync_copy(src_hbm.at[pl.ds(c0, CPT_LAST)],
                        si.at[pl.ds(0, CPT_LAST)])
        pltpu.sync_copy(dst_hbm.at[pl.ds(c0, CPT_LAST)],
                        di.at[pl.ds(0, CPT_LAST)])

    @pl.when(wid != NW - 1)
    def _():
        pltpu.sync_copy(src_hbm.at[pl.ds(c0, CPT)], si)
        pltpu.sync_copy(dst_hbm.at[pl.ds(c0, CPT)], di)

    return lax.select(wid == NW - 1, CPT_LAST, CPT)


def _agg_phase(h_hbm, out_hbm, zeros_hbm, si, di, msgs, acc_sh,
               gsems, ssems, cpt, r0, cid):
    def gather(i, k):
        pltpu.make_async_copy(h_hbm.at[si.at[i]], msgs[k], gsems[k]).start()

    def wait_g(i, k):
        pltpu.make_async_copy(h_hbm.at[si.at[i]], msgs[k], gsems[k]).wait()

    def scat(i, k):
        pltpu.make_async_copy(msgs[k], acc_sh.at[di.at[i]],
                              ssems[k]).start(add=True)

    def wait_s(i, k):
        pltpu.make_async_copy(msgs[k], acc_sh.at[di.at[i]], ssems[k]).wait()

    for k in range(NSLOT):
        gather(k, k)
    pltpu.sync_copy(zeros_hbm.at[pl.ds(r0, RPS)], acc_sh.at[pl.ds(r0, RPS)])
    plsc.subcore_barrier()

    @pl.loop(0, cpt // NSLOT - 1)
    def _(j):
        i = NSLOT * j
        for k in range(NSLOT):
            wait_g(i + k, k)
            scat(i + k, k)
        for k in range(NSLOT):
            wait_s(i + k, k)
            gather(i + NSLOT + k, k)

    last = cpt - NSLOT
    for k in range(NSLOT):
        wait_g(last + k, k)
        scat(last + k, k)
    for k in range(NSLOT):
        wait_s(last + k, k)

    plsc.subcore_barrier()
    pltpu.sync_copy(acc_sh.at[pl.ds(r0, RPS)], out_hbm.at[cid, pl.ds(r0, RPS)])


_AGG_SCRATCH = (
    [pltpu.VMEM((CPT, CHUNK), jnp.int32),
     pltpu.VMEM((CPT, CHUNK), jnp.int32)]
    + [pltpu.VMEM((CHUNK, DH), jnp.float32)] * NSLOT
    + [pltpu.VMEM_SHARED((NP, DH), jnp.float32)]
    + [pltpu.SemaphoreType.DMA] * (2 * NSLOT)
)


@functools.partial(
    pl.kernel,
    out_type=(jax.ShapeDtypeStruct((NC, NP, DH), jnp.float32),
              jax.ShapeDtypeStruct((NC, NP, DH), jnp.float32)),
    mesh=_mesh,
    compiler_params=_sc_params,
    scratch_types=_AGG_SCRATCH,
)
def _agg2x64(h_lo, h_hi, ei_hbm, zeros_hbm, out_lo, out_hi,
             si, di, m0, m1, m2, m3, acc_sh, g0, g1, g2, g3, s0, s1, s2, s3):
    src_hbm = ei_hbm.at[0]
    dst_hbm = ei_hbm.at[1]
    msgs = [m0, m1, m2, m3]
    gsems = [g0, g1, g2, g3]
    ssems = [s0, s1, s2, s3]
    cid = lax.axis_index("c")
    sid = lax.axis_index("s")
    r0 = sid * RPS
    wid = cid * NS + sid
    cpt = _load_idx(src_hbm, dst_hbm, si, di, wid, wid * CPT)
    _agg_phase(h_lo, out_lo, zeros_hbm, si, di, msgs, acc_sh,
               gsems, ssems, cpt, r0, cid)
    plsc.subcore_barrier()
    _agg_phase(h_hi, out_hi, zeros_hbm, si, di, msgs, acc_sh,
               gsems, ssems, cpt, r0, cid)


@functools.partial(
    pl.kernel,
    out_type=jax.ShapeDtypeStruct((NC, NP, DH), jnp.float32),
    mesh=_mesh,
    compiler_params=_sc_params,
    scratch_types=_AGG_SCRATCH,
)
def _agg64(h_hbm, ei_hbm, zeros_hbm, out_hbm,
           si, di, m0, m1, m2, m3, acc_sh, g0, g1, g2, g3, s0, s1, s2, s3):
    src_hbm = ei_hbm.at[0]
    dst_hbm = ei_hbm.at[1]
    msgs = [m0, m1, m2, m3]
    gsems = [g0, g1, g2, g3]
    ssems = [s0, s1, s2, s3]
    cid = lax.axis_index("c")
    sid = lax.axis_index("s")
    r0 = sid * RPS
    wid = cid * NS + sid
    cpt = _load_idx(src_hbm, dst_hbm, si, di, wid, wid * CPT)
    _agg_phase(h_hbm, out_hbm, zeros_hbm, si, di, msgs, acc_sh,
               gsems, ssems, cpt, r0, cid)


def _scale_body(feat, hs, out_lo, out_hi):
    deg = hs[0, 0:N, 0:1] + hs[1, 0:N, 0:1]
    ns = lax.rsqrt(jnp.maximum(deg, 1.0))
    h = feat[...] * ns
    out_lo[0:N, :] = h[:, 0:DH]
    out_hi[0:N, :] = h[:, DH:D_IN]
    zero_tail = jnp.zeros((NP - N, DH), jnp.float32)
    out_lo[N:NP, :] = zero_tail
    out_hi[N:NP, :] = zero_tail


_scale = pl.pallas_call(
    _scale_body,
    out_shape=(jax.ShapeDtypeStruct((NP, DH), jnp.float32),
               jax.ShapeDtypeStruct((NP, DH), jnp.float32)),
)


MMB = 2048


def _mm_body(a_lo, a_hi, hd, hs, w1, b1, w2, out):
    lo = a_lo[0] + a_lo[1]
    hi = a_hi[0] + a_hi[1]
    nd = lax.rsqrt(jnp.maximum(hd[0, :, 0:1] + hd[1, :, 0:1], 1.0))
    t = (jnp.dot(lo * nd, w1[0:DH, :], preferred_element_type=jnp.float32)
         + jnp.dot(hi * nd, w1[DH:D_IN, :], preferred_element_type=jnp.float32)
         + b1[...])
    t = jnp.maximum(t, 0.0)
    ns = lax.rsqrt(jnp.maximum(hs[0, :, 0:1] + hs[1, :, 0:1], 1.0))
    out[...] = jnp.dot(t * ns, w2[...], preferred_element_type=jnp.float32)


_mm = pl.pallas_call(
    _mm_body,
    grid=(NP // MMB,),
    in_specs=[
        pl.BlockSpec((2, MMB, DH), lambda r: (0, r, 0)),
        pl.BlockSpec((2, MMB, DH), lambda r: (0, r, 0)),
        pl.BlockSpec((2, MMB, 16), lambda r: (0, r, 0)),
        pl.BlockSpec((2, MMB, 16), lambda r: (0, r, 0)),
        pl.BlockSpec((D_IN, D_HID), lambda r: (0, 0)),
        pl.BlockSpec((1, D_HID), lambda r: (0, 0)),
        pl.BlockSpec((D_HID, D_OUT), lambda r: (0, 0)),
    ],
    out_specs=pl.BlockSpec((MMB, D_OUT), lambda r: (r, 0)),
    out_shape=jax.ShapeDtypeStruct((NP, D_OUT), jnp.float32),
)


def _final_body(agg, hd, b2, out):
    a = agg[0, 0:N] + agg[1, 0:N]
    nd = lax.rsqrt(jnp.maximum(hd[0, 0:N, 0:1] + hd[1, 0:N, 0:1], 1.0))
    out[...] = a * nd + b2[...]


_final = pl.pallas_call(
    _final_body,
    out_shape=jax.ShapeDtypeStruct((N, D_OUT), jnp.float32),
)


def kernel(features, edge_index, W1, b1, W2, b2):
    ei3 = edge_index.astype(jnp.int32).reshape(2, NCHUNK, CHUNK)
    ones16 = jnp.ones((CHUNK, 16), jnp.float32)
    z16 = jnp.zeros((NP, 16), jnp.float32)
    z64 = jnp.zeros((NP, DH), jnp.float32)

    hsrc, hdst = _deg_kernel(ei3, ones16, z16)
    h_lo, h_hi = _scale(features, hsrc)
    a_lo, a_hi = _agg2x64(h_lo, h_hi, ei3, z64)
    z = _mm(a_lo, a_hi, hdst, hsrc, W1, b1.reshape(1, D_HID), W2)
    agg2 = _agg64(z, ei3, z64)
    return _final(agg2, hdst, b2.reshape(1, D_OUT))


# speedup vs baseline: 1.1670x; 1.0155x over previous
import functools

import jax
import jax.numpy as jnp
from jax import lax
from jax.experimental import pallas as pl
from jax.experimental.pallas import tpu as pltpu
from jax.experimental.pallas import tpu_sc as plsc

N = 10000
NP = 10240
E = 320000
D_IN = 128
D_HID = 128
D_OUT = 64
DH = 64

NC = 2
NS = 16
NW = NC * NS
CHUNK = 128
NCHUNK = E // CHUNK
CPT = 80
CPT_LAST = NCHUNK - (NW - 1) * CPT
RPS = NP // NS

_mesh = plsc.VectorSubcoreMesh(core_axis_name="c", subcore_axis_name="s")
_sc_params = pltpu.CompilerParams(use_tc_tiling_on_sc=False)


@functools.partial(
    pl.kernel,
    out_type=(jax.ShapeDtypeStruct((NC, NP, 16), jnp.float32),
              jax.ShapeDtypeStruct((NC, NP, 16), jnp.float32)),
    mesh=_mesh,
    compiler_params=_sc_params,
    scratch_types=[
        pltpu.VMEM((CPT, CHUNK), jnp.int32),
        pltpu.VMEM((CPT, CHUNK), jnp.int32),
        pltpu.VMEM((CHUNK, 16), jnp.float32),
        pltpu.VMEM_SHARED((NP, 16), jnp.float32),
        pltpu.VMEM_SHARED((NP, 16), jnp.float32),
        pltpu.SemaphoreType.DMA,
        pltpu.SemaphoreType.DMA,
        pltpu.SemaphoreType.DMA,
        pltpu.SemaphoreType.DMA,
        pltpu.SemaphoreType.DMA,
        pltpu.SemaphoreType.DMA,
        pltpu.SemaphoreType.DMA,
        pltpu.SemaphoreType.DMA,
        pltpu.SemaphoreType.DMA,
        pltpu.SemaphoreType.DMA,
    ],
)
def _deg_kernel(ei_hbm, ones_hbm, zeros_hbm,
                hsrc_out, hdst_out, si, di, ones_v, hs_sh, hd_sh,
                a0, a1, a2, a3, a4, b0, b1, b2, b3, b4):
    src_hbm = ei_hbm.at[0]
    dst_hbm = ei_hbm.at[1]
    asems = [a0, a1, a2, a3, a4]
    bsems = [b0, b1, b2, b3, b4]
    cid = lax.axis_index("c")
    sid = lax.axis_index("s")
    r0 = sid * RPS
    pltpu.sync_copy(zeros_hbm.at[pl.ds(r0, RPS)], hs_sh.at[pl.ds(r0, RPS)])
    pltpu.sync_copy(zeros_hbm.at[pl.ds(r0, RPS)], hd_sh.at[pl.ds(r0, RPS)])
    pltpu.sync_copy(ones_hbm, ones_v)
    wid = cid * NS + sid
    c0 = wid * CPT
    cpt = _load_idx(src_hbm, dst_hbm, si, di, wid, c0)
    plsc.subcore_barrier()

    def s_hs(i, k):
        pltpu.make_async_copy(ones_v, hs_sh.at[si.at[i]],
                              asems[k]).start(add=True)

    def w_hs(i, k):
        pltpu.make_async_copy(ones_v, hs_sh.at[si.at[i]], asems[k]).wait()

    def s_hd(i, k):
        pltpu.make_async_copy(ones_v, hd_sh.at[di.at[i]],
                              bsems[k]).start(add=True)

    def w_hd(i, k):
        pltpu.make_async_copy(ones_v, hd_sh.at[di.at[i]], bsems[k]).wait()

    for k in range(NSLOT):
        s_hs(k, k)
        s_hd(k, k)

    @pl.loop(0, cpt // NSLOT - 1)
    def _(j):
        i = NSLOT * j
        for k in range(NSLOT):
            w_hs(i + k, k)
            s_hs(i + NSLOT + k, k)
            w_hd(i + k, k)
            s_hd(i + NSLOT + k, k)

    last = cpt - NSLOT
    for k in range(NSLOT):
        w_hs(last + k, k)
        w_hd(last + k, k)

    plsc.subcore_barrier()
    pltpu.sync_copy(hs_sh.at[pl.ds(r0, RPS)], hsrc_out.at[cid, pl.ds(r0, RPS)])
    pltpu.sync_copy(hd_sh.at[pl.ds(r0, RPS)], hdst_out.at[cid, pl.ds(r0, RPS)])


NSLOT = 5


def _load_idx(src_hbm, dst_hbm, si, di, wid, c0):
    @pl.when(wid == NW - 1)
    def _():
        pltpu.sync_copy(src_hbm.at[pl.ds(c0, CPT_LAST)],
                        si.at[pl.ds(0, CPT_LAST)])
        pltpu.sync_copy(dst_hbm.at[pl.ds(c0, CPT_LAST)],
                        di.at[pl.ds(0, CPT_LAST)])

    @pl.when(wid != NW - 1)
    def _():
        pltpu.sync_copy(src_hbm.at[pl.ds(c0, CPT)], si)
        pltpu.sync_copy(dst_hbm.at[pl.ds(c0, CPT)], di)

    return lax.select(wid == NW - 1, CPT_LAST, CPT)


def _agg_phase(h_hbm, out_hbm, zeros_hbm, si, di, msgs, acc_sh,
               gsems, ssems, cpt, r0, cid):
    def gather(i, k):
        pltpu.make_async_copy(h_hbm.at[si.at[i]], msgs[k], gsems[k]).start()

    def wait_g(i, k):
        pltpu.make_async_copy(h_hbm.at[si.at[i]], msgs[k], gsems[k]).wait()

    def scat(i, k):
        pltpu.make_async_copy(msgs[k], acc_sh.at[di.at[i]],
                              ssems[k]).start(add=True)

    def wait_s(i, k):
        pltpu.make_async_copy(msgs[k], acc_sh.at[di.at[i]], ssems[k]).wait()

    for k in range(NSLOT):
        gather(k, k)
    pltpu.sync_copy(zeros_hbm.at[pl.ds(r0, RPS)], acc_sh.at[pl.ds(r0, RPS)])
    plsc.subcore_barrier()

    @pl.loop(0, cpt // NSLOT - 1)
    def _(j):
        i = NSLOT * j
        for k in range(NSLOT):
            wait_g(i + k, k)
            scat(i + k, k)
        for k in range(NSLOT):
            wait_s(i + k, k)
            gather(i + NSLOT + k, k)

    last = cpt - NSLOT
    for k in range(NSLOT):
        wait_g(last + k, k)
        scat(last + k, k)
    for k in range(NSLOT):
        wait_s(last + k, k)

    plsc.subcore_barrier()
    pltpu.sync_copy(acc_sh.at[pl.ds(r0, RPS)], out_hbm.at[cid, pl.ds(r0, RPS)])


_AGG_SCRATCH = (
    [pltpu.VMEM((CPT, CHUNK), jnp.int32),
     pltpu.VMEM((CPT, CHUNK), jnp.int32)]
    + [pltpu.VMEM((CHUNK, DH), jnp.float32)] * NSLOT
    + [pltpu.VMEM_SHARED((NP, DH), jnp.float32)]
    + [pltpu.SemaphoreType.DMA] * (2 * NSLOT)
)


@functools.partial(
    pl.kernel,
    out_type=(jax.ShapeDtypeStruct((NC, NP, DH), jnp.float32),
              jax.ShapeDtypeStruct((NC, NP, DH), jnp.float32)),
    mesh=_mesh,
    compiler_params=_sc_params,
    scratch_types=_AGG_SCRATCH,
)
def _agg2x64(h_lo, h_hi, ei_hbm, zeros_hbm, out_lo, out_hi,
             si, di, m0, m1, m2, m3, m4, acc_sh,
             g0, g1, g2, g3, g4, s0, s1, s2, s3, s4):
    src_hbm = ei_hbm.at[0]
    dst_hbm = ei_hbm.at[1]
    msgs = [m0, m1, m2, m3, m4]
    gsems = [g0, g1, g2, g3, g4]
    ssems = [s0, s1, s2, s3, s4]
    cid = lax.axis_index("c")
    sid = lax.axis_index("s")
    r0 = sid * RPS
    wid = cid * NS + sid
    cpt = _load_idx(src_hbm, dst_hbm, si, di, wid, wid * CPT)
    _agg_phase(h_lo, out_lo, zeros_hbm, si, di, msgs, acc_sh,
               gsems, ssems, cpt, r0, cid)
    plsc.subcore_barrier()
    _agg_phase(h_hi, out_hi, zeros_hbm, si, di, msgs, acc_sh,
               gsems, ssems, cpt, r0, cid)


@functools.partial(
    pl.kernel,
    out_type=jax.ShapeDtypeStruct((NC, NP, DH), jnp.float32),
    mesh=_mesh,
    compiler_params=_sc_params,
    scratch_types=_AGG_SCRATCH,
)
def _agg64(h_hbm, ei_hbm, zeros_hbm, out_hbm,
           si, di, m0, m1, m2, m3, m4, acc_sh,
           g0, g1, g2, g3, g4, s0, s1, s2, s3, s4):
    src_hbm = ei_hbm.at[0]
    dst_hbm = ei_hbm.at[1]
    msgs = [m0, m1, m2, m3, m4]
    gsems = [g0, g1, g2, g3, g4]
    ssems = [s0, s1, s2, s3, s4]
    cid = lax.axis_index("c")
    sid = lax.axis_index("s")
    r0 = sid * RPS
    wid = cid * NS + sid
    cpt = _load_idx(src_hbm, dst_hbm, si, di, wid, wid * CPT)
    _agg_phase(h_hbm, out_hbm, zeros_hbm, si, di, msgs, acc_sh,
               gsems, ssems, cpt, r0, cid)


def _scale_body(feat, hs, out_lo, out_hi):
    deg = hs[0, 0:N, 0:1] + hs[1, 0:N, 0:1]
    ns = lax.rsqrt(jnp.maximum(deg, 1.0))
    h = feat[...] * ns
    out_lo[0:N, :] = h[:, 0:DH]
    out_hi[0:N, :] = h[:, DH:D_IN]
    zero_tail = jnp.zeros((NP - N, DH), jnp.float32)
    out_lo[N:NP, :] = zero_tail
    out_hi[N:NP, :] = zero_tail


_scale = pl.pallas_call(
    _scale_body,
    out_shape=(jax.ShapeDtypeStruct((NP, DH), jnp.float32),
               jax.ShapeDtypeStruct((NP, DH), jnp.float32)),
)


MMB = 2048


def _mm_body(a_lo, a_hi, hd, hs, w1, b1, w2, out):
    lo = a_lo[0] + a_lo[1]
    hi = a_hi[0] + a_hi[1]
    nd = lax.rsqrt(jnp.maximum(hd[0, :, 0:1] + hd[1, :, 0:1], 1.0))
    t = (jnp.dot(lo * nd, w1[0:DH, :], preferred_element_type=jnp.float32)
         + jnp.dot(hi * nd, w1[DH:D_IN, :], preferred_element_type=jnp.float32)
         + b1[...])
    t = jnp.maximum(t, 0.0)
    ns = lax.rsqrt(jnp.maximum(hs[0, :, 0:1] + hs[1, :, 0:1], 1.0))
    out[...] = jnp.dot(t * ns, w2[...], preferred_element_type=jnp.float32)


_mm = pl.pallas_call(
    _mm_body,
    grid=(NP // MMB,),
    in_specs=[
        pl.BlockSpec((2, MMB, DH), lambda r: (0, r, 0)),
        pl.BlockSpec((2, MMB, DH), lambda r: (0, r, 0)),
        pl.BlockSpec((2, MMB, 16), lambda r: (0, r, 0)),
        pl.BlockSpec((2, MMB, 16), lambda r: (0, r, 0)),
        pl.BlockSpec((D_IN, D_HID), lambda r: (0, 0)),
        pl.BlockSpec((1, D_HID), lambda r: (0, 0)),
        pl.BlockSpec((D_HID, D_OUT), lambda r: (0, 0)),
    ],
    out_specs=pl.BlockSpec((MMB, D_OUT), lambda r: (r, 0)),
    out_shape=jax.ShapeDtypeStruct((NP, D_OUT), jnp.float32),
)


def _final_body(agg, hd, b2, out):
    a = agg[0, 0:N] + agg[1, 0:N]
    nd = lax.rsqrt(jnp.maximum(hd[0, 0:N, 0:1] + hd[1, 0:N, 0:1], 1.0))
    out[...] = a * nd + b2[...]


_final = pl.pallas_call(
    _final_body,
    out_shape=jax.ShapeDtypeStruct((N, D_OUT), jnp.float32),
)


def kernel(features, edge_index, W1, b1, W2, b2):
    ei3 = edge_index.astype(jnp.int32).reshape(2, NCHUNK, CHUNK)
    ones16 = jnp.ones((CHUNK, 16), jnp.float32)
    z16 = jnp.zeros((NP, 16), jnp.float32)
    z64 = jnp.zeros((NP, DH), jnp.float32)

    hsrc, hdst = _deg_kernel(ei3, ones16, z16)
    h_lo, h_hi = _scale(features, hsrc)
    a_lo, a_hi = _agg2x64(h_lo, h_hi, ei3, z64)
    z = _mm(a_lo, a_hi, hdst, hsrc, W1, b1.reshape(1, D_HID), W2)
    agg2 = _agg64(z, ei3, z64)
    return _final(agg2, hdst, b2.reshape(1, D_OUT))
